# baseline (speedup 1.0000x reference)
import functools

import jax
import jax.numpy as jnp
from jax import lax
from jax.experimental import pallas as pl
from jax.experimental.pallas import tpu as pltpu
from jax.experimental.pallas import tpu_sc as plsc

BATCH = 4
SEQ = 4096
EMBED = 1024
LANES = 16
VECS_PER_ROW = EMBED // LANES

NUM_CORES = 2
NUM_SUBCORES = 16
NUM_WORKERS = NUM_CORES * NUM_SUBCORES

SEQ_PER_WORKER = SEQ // NUM_WORKERS
CHUNK = 8
NUM_CHUNKS = SEQ_PER_WORKER // CHUNK
NBUF = 3
DEPTH = 1


def _sc_body(emb_hbm, table_hbm, out_hbm, *scratch):
    tbufs = scratch[0:NBUF]
    bufs = scratch[NBUF:2 * NBUF]
    isems = scratch[2 * NBUF:3 * NBUF]
    osems = scratch[3 * NBUF:4 * NBUF]

    cid = lax.axis_index("c")
    sid = lax.axis_index("s")
    wid = sid * NUM_CORES + cid
    seq_base = wid * SEQ_PER_WORKER

    def load_descs(g, slot):
        s0 = seq_base + g * CHUNK
        return [
            pltpu.make_async_copy(
                table_hbm.at[pl.ds(s0, CHUNK)], tbufs[slot], isems[slot]),
            pltpu.make_async_copy(
                emb_hbm.at[:, pl.ds(s0, CHUNK)], bufs[slot], isems[slot]),
        ]

    def store_descs(g, slot):
        s0 = seq_base + g * CHUNK
        return [pltpu.make_async_copy(
            bufs[slot], out_hbm.at[:, pl.ds(s0, CHUNK)], osems[slot])]

    def add_chunk(slot):
        buf, tbuf = bufs[slot], tbufs[slot]

        def row_step(r, _):
            for j in range(VECS_PER_ROW):
                x = tbuf[r, pl.ds(j * LANES, LANES)]
                for b in range(BATCH):
                    plsc.addupdate(buf.at[b, r, pl.ds(j * LANES, LANES)], x)
            return _

        lax.fori_loop(0, CHUNK, row_step, None)

    def issue_loads(g, slot):
        for d in load_descs(g, slot):
            d.start()

    for g in range(DEPTH):
        issue_loads(g, g % NBUF)

    def chunk_body(g, slot, prefetch, guard_drain):
        pre = (slot + DEPTH) % NBUF
        for d in load_descs(g, slot):
            d.wait()
        if prefetch:
            def _drain_and_load():
                for d in store_descs(g, pre):
                    d.wait()
                issue_loads(g + DEPTH, pre)

            if guard_drain:
                @pl.when(g >= NBUF - DEPTH)
                def _gd():
                    _drain_and_load()

                @pl.when(g < NBUF - DEPTH)
                def _ld():
                    issue_loads(g + DEPTH, pre)
            else:
                _drain_and_load()
        add_chunk(slot)
        for d in store_descs(g, slot):
            d.start()

    def group_step(i, _):
        for slot in range(NBUF):
            g = NBUF * i + slot
            chunk_body(g, slot, prefetch=True, guard_drain=True)
        return _

    n_groups = (NUM_CHUNKS - DEPTH - 1) // NBUF
    lax.fori_loop(0, n_groups, group_step, None)
    for g in range(n_groups * NBUF, NUM_CHUNKS):
        chunk_body(g, g % NBUF, prefetch=(g + DEPTH < NUM_CHUNKS),
                   guard_drain=False)
    for slot in range(NBUF):
        for d in store_descs(0, slot):
            d.wait()


@jax.jit
def _run(emb, embed_weight):
    mesh = plsc.VectorSubcoreMesh(
        core_axis_name="c", subcore_axis_name="s",
        num_cores=NUM_CORES, num_subcores=NUM_SUBCORES,
    )
    return pl.kernel(
        _sc_body,
        out_type=jax.ShapeDtypeStruct((BATCH, SEQ, EMBED), jnp.float32),
        mesh=mesh,
        scratch_types=(
            [pltpu.VMEM((CHUNK, EMBED), jnp.float32) for _ in range(NBUF)]
            + [pltpu.VMEM((BATCH, CHUNK, EMBED), jnp.float32)
               for _ in range(NBUF)]
            + [pltpu.SemaphoreType.DMA for _ in range(2 * NBUF)]
        ),
    )(emb, embed_weight)


def kernel(emb, embed_weight):
    return _run(emb, embed_weight)


# speedup vs baseline: 1.0068x; 1.0068x over previous
import functools

import jax
import jax.numpy as jnp
from jax import lax
from jax.experimental import pallas as pl
from jax.experimental.pallas import tpu as pltpu
from jax.experimental.pallas import tpu_sc as plsc

BATCH = 4
SEQ = 4096
EMBED = 1024
LANES = 16
VECS_PER_ROW = EMBED // LANES

NUM_CORES = 2
NUM_SUBCORES = 16
NUM_WORKERS = NUM_CORES * NUM_SUBCORES

SEQ_PER_WORKER = SEQ // NUM_WORKERS
CHUNK = 4
NUM_CHUNKS = SEQ_PER_WORKER // CHUNK
NBUF = 4
DEPTH = 2


def _sc_body(emb_hbm, table_hbm, out_hbm, *scratch):
    tbufs = scratch[0:NBUF]
    bufs = scratch[NBUF:2 * NBUF]
    isems = scratch[2 * NBUF:3 * NBUF]
    osems = scratch[3 * NBUF:4 * NBUF]

    cid = lax.axis_index("c")
    sid = lax.axis_index("s")
    wid = sid * NUM_CORES + cid
    seq_base = wid * SEQ_PER_WORKER

    def load_descs(g, slot):
        s0 = seq_base + g * CHUNK
        return [
            pltpu.make_async_copy(
                table_hbm.at[pl.ds(s0, CHUNK)], tbufs[slot], isems[slot]),
            pltpu.make_async_copy(
                emb_hbm.at[:, pl.ds(s0, CHUNK)], bufs[slot], isems[slot]),
        ]

    def store_descs(g, slot):
        s0 = seq_base + g * CHUNK
        return [pltpu.make_async_copy(
            bufs[slot], out_hbm.at[:, pl.ds(s0, CHUNK)], osems[slot])]

    def add_chunk(slot):
        buf, tbuf = bufs[slot], tbufs[slot]

        def row_step(r, _):
            for j in range(VECS_PER_ROW):
                x = tbuf[r, pl.ds(j * LANES, LANES)]
                for b in range(BATCH):
                    plsc.addupdate(buf.at[b, r, pl.ds(j * LANES, LANES)], x)
            return _

        lax.fori_loop(0, CHUNK, row_step, None)

    def issue_loads(g, slot):
        for d in load_descs(g, slot):
            d.start()

    for g in range(DEPTH):
        issue_loads(g, g % NBUF)

    def chunk_body(g, slot, prefetch, guard_drain):
        pre = (slot + DEPTH) % NBUF
        for d in load_descs(g, slot):
            d.wait()
        if prefetch:
            def _drain_and_load():
                for d in store_descs(g, pre):
                    d.wait()
                issue_loads(g + DEPTH, pre)

            if guard_drain:
                @pl.when(g >= NBUF - DEPTH)
                def _gd():
                    _drain_and_load()

                @pl.when(g < NBUF - DEPTH)
                def _ld():
                    issue_loads(g + DEPTH, pre)
            else:
                _drain_and_load()
        add_chunk(slot)
        for d in store_descs(g, slot):
            d.start()

    def group_step(i, _):
        for slot in range(NBUF):
            g = NBUF * i + slot
            chunk_body(g, slot, prefetch=True, guard_drain=True)
        return _

    n_groups = (NUM_CHUNKS - DEPTH - 1) // NBUF
    lax.fori_loop(0, n_groups, group_step, None)
    for g in range(n_groups * NBUF, NUM_CHUNKS):
        chunk_body(g, g % NBUF, prefetch=(g + DEPTH < NUM_CHUNKS),
                   guard_drain=False)
    for slot in range(NBUF):
        for d in store_descs(0, slot):
            d.wait()


@jax.jit
def _run(emb, embed_weight):
    mesh = plsc.VectorSubcoreMesh(
        core_axis_name="c", subcore_axis_name="s",
        num_cores=NUM_CORES, num_subcores=NUM_SUBCORES,
    )
    return pl.kernel(
        _sc_body,
        out_type=jax.ShapeDtypeStruct((BATCH, SEQ, EMBED), jnp.float32),
        mesh=mesh,
        scratch_types=(
            [pltpu.VMEM((CHUNK, EMBED), jnp.float32) for _ in range(NBUF)]
            + [pltpu.VMEM((BATCH, CHUNK, EMBED), jnp.float32)
               for _ in range(NBUF)]
            + [pltpu.SemaphoreType.DMA for _ in range(2 * NBUF)]
        ),
    )(emb, embed_weight)


def kernel(emb, embed_weight):
    return _run(emb, embed_weight)


# speedup vs baseline: 1.0655x; 1.0582x over previous
import functools

import jax
import jax.numpy as jnp
from jax import lax
from jax.experimental import pallas as pl
from jax.experimental.pallas import tpu as pltpu
from jax.experimental.pallas import tpu_sc as plsc

BATCH = 4
SEQ = 4096
EMBED = 1024
LANES = 16
VECS_PER_ROW = EMBED // LANES

NUM_CORES = 2
NUM_SUBCORES = 16
NUM_WORKERS = NUM_CORES * NUM_SUBCORES

SEQ_PER_WORKER = SEQ // NUM_WORKERS
CHUNK = 4
NUM_CHUNKS = SEQ_PER_WORKER // CHUNK
NBUF = 4
DEPTH = 2


def _sc_body(emb_hbm, table_hbm, out_hbm, *scratch):
    tbufs = scratch[0:NBUF]
    bufs = scratch[NBUF:2 * NBUF]
    isems = scratch[2 * NBUF:3 * NBUF]
    osems = scratch[3 * NBUF:4 * NBUF]

    cid = lax.axis_index("c")
    sid = lax.axis_index("s")
    wid = sid * NUM_CORES + cid
    seq_base = wid * SEQ_PER_WORKER

    def load_descs(g, slot):
        s0 = seq_base + g * CHUNK
        return [
            pltpu.make_async_copy(
                table_hbm.at[pl.ds(s0, CHUNK)], tbufs[slot], isems[slot]),
            pltpu.make_async_copy(
                emb_hbm.at[:, pl.ds(s0, CHUNK)], bufs[slot], isems[slot]),
        ]

    def store_descs(g, slot):
        s0 = seq_base + g * CHUNK
        return [pltpu.make_async_copy(
            bufs[slot], out_hbm.at[:, pl.ds(s0, CHUNK)], osems[slot])]

    def add_chunk(slot):
        buf, tbuf = bufs[slot], tbufs[slot]

        def row_step(r, _):
            for j in range(VECS_PER_ROW):
                x = tbuf[r, pl.ds(j * LANES, LANES)]
                for b in range(BATCH):
                    plsc.addupdate(buf.at[b, r, pl.ds(j * LANES, LANES)], x)
            return _

        lax.fori_loop(0, CHUNK, row_step, None)

    def issue_loads(g, slot):
        for d in load_descs(g, slot):
            d.start()

    for g in range(DEPTH):
        issue_loads(g, g % NBUF)

    def group_step(i, _):
        for slot in range(NBUF):
            g = NBUF * i + slot
            pre = (slot + DEPTH) % NBUF
            for d in load_descs(g, slot):
                d.wait()

            @pl.when(g + DEPTH < NUM_CHUNKS)
            def _prefetch():
                @pl.when(g >= NBUF - DEPTH)
                def _drain():
                    for d in store_descs(g, pre):
                        d.wait()

                issue_loads(g + DEPTH, pre)

            add_chunk(slot)
            for d in store_descs(g, slot):
                d.start()
        return _

    assert NUM_CHUNKS % NBUF == 0
    lax.fori_loop(0, NUM_CHUNKS // NBUF, group_step, None)
    for slot in range(NBUF):
        for d in store_descs(0, slot):
            d.wait()


@jax.jit
def _run(emb, embed_weight):
    mesh = plsc.VectorSubcoreMesh(
        core_axis_name="c", subcore_axis_name="s",
        num_cores=NUM_CORES, num_subcores=NUM_SUBCORES,
    )
    return pl.kernel(
        _sc_body,
        out_type=jax.ShapeDtypeStruct((BATCH, SEQ, EMBED), jnp.float32),
        mesh=mesh,
        scratch_types=(
            [pltpu.VMEM((CHUNK, EMBED), jnp.float32) for _ in range(NBUF)]
            + [pltpu.VMEM((BATCH, CHUNK, EMBED), jnp.float32)
               for _ in range(NBUF)]
            + [pltpu.SemaphoreType.DMA for _ in range(2 * NBUF)]
        ),
    )(emb, embed_weight)


def kernel(emb, embed_weight):
    return _run(emb, embed_weight)
